# Initial kernel scaffold; baseline (speedup 1.0000x reference)
import functools

import jax
import jax.numpy as jnp
from jax import lax
from jax.experimental import pallas as pl
from jax.experimental.pallas import tpu as pltpu
from jax.experimental.pallas import tpu_sc as plsc

_BATCH = 4096
_LEN = 200
_DIM = 128
_NCLS = 2
_NWORKERS = 32
_SPW = _BATCH // _NWORKERS


def _sc_body(xT_hbm, table_hbm, w_hbm, out_hbm,
             idx_v, rows_v, w_v, acc_v, out_v, sem0, sem1):
    wid = lax.axis_index("s") * 2 + lax.axis_index("c")
    base = wid * _SPW

    pltpu.sync_copy(xT_hbm.at[:, pl.ds(base, _SPW)], idx_v)
    pltpu.sync_copy(w_hbm, w_v)

    zeros16 = jnp.zeros((16,), jnp.float32)

    def zbody(t, carry):
        acc_v[t, 0] = zeros16
        acc_v[t, 1] = zeros16
        return carry
    lax.fori_loop(0, _SPW, zbody, 0)

    sems = (sem0, sem1)

    def gcopy(l, u):
        return pltpu.make_async_copy(
            table_hbm.at[idx_v.at[l]], rows_v.at[u], sems[u])

    gcopy(0, 0).start()
    gcopy(1, 1).start()

    def compute(l, u):
        w0 = [w_v[0, l, pl.ds(16 * k, 16)] for k in range(8)]
        w1 = [w_v[1, l, pl.ds(16 * k, 16)] for k in range(8)]

        def tbody(t, carry):
            r = [rows_v[u, t, pl.ds(16 * k, 16)] for k in range(8)]
            d0 = r[0] * w0[0]
            d1 = r[0] * w1[0]
            for k in range(1, 8):
                d0 = d0 + r[k] * w0[k]
                d1 = d1 + r[k] * w1[k]
            plsc.addupdate(acc_v.at[t, 0], d0)
            plsc.addupdate(acc_v.at[t, 1], d1)
            return carry
        lax.fori_loop(0, _SPW, tbody, 0)

    def pair_body(i, carry):
        for u in range(2):
            l = 2 * i + u
            gcopy(l, u).wait()

            @pl.when(l + 2 < _LEN)
            def _():
                gcopy(l + 2, u).start()

            compute(l, u)
        return carry
    lax.fori_loop(0, _LEN // 2, pair_body, 0)

    def fbody(t, carry):
        out_v[t, 0] = jnp.sum(acc_v[t, 0])
        out_v[t, 1] = jnp.sum(acc_v[t, 1])
        return carry
    lax.fori_loop(0, _SPW, fbody, 0)

    pltpu.sync_copy(out_v, out_hbm.at[pl.ds(base, _SPW)])


_sc_embed = functools.partial(
    pl.kernel,
    out_type=jax.ShapeDtypeStruct((_BATCH, _NCLS), jnp.float32),
    mesh=plsc.VectorSubcoreMesh(core_axis_name="c", subcore_axis_name="s"),
    scratch_types=[
        pltpu.VMEM((_LEN, _SPW), jnp.int32),
        pltpu.VMEM((2, _SPW, _DIM), jnp.float32),
        pltpu.VMEM((_NCLS, _LEN, _DIM), jnp.float32),
        pltpu.VMEM((_SPW, _NCLS, 16), jnp.float32),
        pltpu.VMEM((_SPW, _NCLS), jnp.float32),
        pltpu.SemaphoreType.DMA,
        pltpu.SemaphoreType.DMA,
    ],
)(_sc_body)


def _logsoftmax_body(z_ref, b_ref, o_ref):
    z = z_ref[...] + b_ref[...]
    m = jnp.max(z, axis=-1, keepdims=True)
    e = jnp.exp(z - m)
    lse = m + jnp.log(jnp.sum(e, axis=-1, keepdims=True))
    o_ref[...] = z - lse


def kernel(x, table, W, b):
    xT = x.T
    Wr = W.reshape(_NCLS, _LEN, _DIM)
    logits = _sc_embed(xT, table, Wr)
    return pl.pallas_call(
        _logsoftmax_body,
        out_shape=jax.ShapeDtypeStruct((_BATCH, _NCLS), jnp.float32),
    )(logits, b.reshape(1, _NCLS))


# speedup vs baseline: 7.3650x; 7.3650x over previous
import functools

import jax
import jax.numpy as jnp
from jax import lax
from jax.experimental import pallas as pl
from jax.experimental.pallas import tpu as pltpu
from jax.experimental.pallas import tpu_sc as plsc

_BATCH = 4096
_LEN = 200
_DIM = 128
_NCLS = 2
_NWORKERS = 32
_SPW = _BATCH // _NWORKERS


def _sc_body(xT_hbm, table_hbm, w_hbm, out_hbm,
             idx_v, rows_v, w_v, acc_v, semr0, semr1, semw0, semw1):
    wid = lax.axis_index("s") * 2 + lax.axis_index("c")
    base = wid * _SPW

    pltpu.sync_copy(xT_hbm.at[:, pl.ds(base, _SPW)], idx_v)

    zeros16 = jnp.zeros((16,), jnp.float32)

    def zbody(t, carry):
        acc_v[t, 0] = zeros16
        acc_v[t, 1] = zeros16
        return carry
    lax.fori_loop(0, _SPW, zbody, 0)

    semr = (semr0, semr1)
    semw = (semw0, semw1)

    def gcopy(l, u):
        return pltpu.make_async_copy(
            table_hbm.at[idx_v.at[l]], rows_v.at[u], semr[u])

    def wcopy(l, u):
        return pltpu.make_async_copy(w_hbm.at[:, l, :], w_v.at[u], semw[u])

    gcopy(0, 0).start()
    wcopy(0, 0).start()
    gcopy(1, 1).start()
    wcopy(1, 1).start()

    def compute(u):
        w0 = [w_v[u, 0, pl.ds(16 * k, 16)] for k in range(8)]
        w1 = [w_v[u, 1, pl.ds(16 * k, 16)] for k in range(8)]

        def tbody(t, carry):
            r = [rows_v[u, t, pl.ds(16 * k, 16)] for k in range(8)]
            d0 = r[0] * w0[0]
            d1 = r[0] * w1[0]
            for k in range(1, 8):
                d0 = d0 + r[k] * w0[k]
                d1 = d1 + r[k] * w1[k]
            plsc.addupdate(acc_v.at[t, 0], d0)
            plsc.addupdate(acc_v.at[t, 1], d1)
            return carry
        lax.fori_loop(0, _SPW, tbody, 0)

    def pair_body(i, carry):
        for u in range(2):
            l = 2 * i + u
            gcopy(l, u).wait()
            wcopy(l, u).wait()
            compute(u)

            @pl.when(l + 2 < _LEN)
            def _():
                gcopy(l + 2, u).start()
                wcopy(l + 2, u).start()
        return carry
    lax.fori_loop(0, _LEN // 2, pair_body, 0)

    pltpu.sync_copy(acc_v, out_hbm.at[pl.ds(base, _SPW)])


_sc_embed = functools.partial(
    pl.kernel,
    out_type=jax.ShapeDtypeStruct((_BATCH, _NCLS, 16), jnp.float32),
    mesh=plsc.VectorSubcoreMesh(core_axis_name="c", subcore_axis_name="s"),
    scratch_types=[
        pltpu.VMEM((_LEN, _SPW), jnp.int32),
        pltpu.VMEM((2, _SPW, _DIM), jnp.float32),
        pltpu.VMEM((2, _NCLS, _DIM), jnp.float32),
        pltpu.VMEM((_SPW, _NCLS, 16), jnp.float32),
        pltpu.SemaphoreType.DMA,
        pltpu.SemaphoreType.DMA,
        pltpu.SemaphoreType.DMA,
        pltpu.SemaphoreType.DMA,
    ],
)(_sc_body)


def _logsoftmax_body(p_ref, b_ref, o_ref):
    p = p_ref[...]
    z0 = jnp.sum(p[:, :16], axis=1, keepdims=True)
    z1 = jnp.sum(p[:, 16:], axis=1, keepdims=True)
    z = jnp.concatenate([z0, z1], axis=1) + b_ref[...]
    m = jnp.max(z, axis=-1, keepdims=True)
    e = jnp.exp(z - m)
    lse = m + jnp.log(jnp.sum(e, axis=-1, keepdims=True))
    o_ref[...] = z - lse


def kernel(x, table, W, b):
    xT = x.T
    Wr = W.reshape(_NCLS, _LEN, _DIM)
    partials = _sc_embed(xT, table, Wr)
    return pl.pallas_call(
        _logsoftmax_body,
        out_shape=jax.ShapeDtypeStruct((_BATCH, _NCLS), jnp.float32),
    )(partials.reshape(_BATCH, _NCLS * 16), b.reshape(1, _NCLS))


# speedup vs baseline: 7.6573x; 1.0397x over previous
import functools

import jax
import jax.numpy as jnp
from jax import lax
from jax.experimental import pallas as pl
from jax.experimental.pallas import tpu as pltpu
from jax.experimental.pallas import tpu_sc as plsc

_BATCH = 4096
_LEN = 200
_DIM = 128
_NCLS = 2
_NWORKERS = 32
_SPW = _BATCH // _NWORKERS


def _sc_body(xT_hbm, table_hbm, w_hbm, out_hbm,
             idx_v, rows_v, w_v, acc_v, semr0, semr1, semw0, semw1):
    wid = lax.axis_index("s") * 2 + lax.axis_index("c")
    base = wid * _SPW

    pltpu.sync_copy(xT_hbm.at[:, pl.ds(base, _SPW)], idx_v)

    zeros16 = jnp.zeros((16,), jnp.float32)

    def zbody(t, carry):
        acc_v[t, 0] = zeros16
        acc_v[t, 1] = zeros16
        return carry
    lax.fori_loop(0, _SPW, zbody, 0)

    semr = (semr0, semr1)
    semw = (semw0, semw1)

    def gcopy(l, u):
        return pltpu.make_async_copy(
            table_hbm.at[idx_v.at[l]], rows_v.at[u], semr[u])

    def wcopy(l, u):
        return pltpu.make_async_copy(w_hbm.at[:, l, :], w_v.at[u], semw[u])

    gcopy(0, 0).start()
    wcopy(0, 0).start()
    gcopy(1, 1).start()
    wcopy(1, 1).start()

    def compute(u):
        w0 = [w_v[u, 0, pl.ds(16 * k, 16)] for k in range(8)]
        w1 = [w_v[u, 1, pl.ds(16 * k, 16)] for k in range(8)]

        def tbody(t, carry):
            r = [rows_v[u, t, pl.ds(16 * k, 16)] for k in range(8)]
            d0 = r[0] * w0[0]
            d1 = r[0] * w1[0]
            for k in range(1, 8):
                d0 = d0 + r[k] * w0[k]
                d1 = d1 + r[k] * w1[k]
            plsc.addupdate(acc_v.at[t, 0], d0)
            plsc.addupdate(acc_v.at[t, 1], d1)
            return carry
        lax.fori_loop(0, _SPW, tbody, 0, unroll=8)

    def pair_body(i, carry):
        for u in range(2):
            l = 2 * i + u
            gcopy(l, u).wait()
            wcopy(l, u).wait()
            compute(u)

            @pl.when(l + 2 < _LEN)
            def _():
                gcopy(l + 2, u).start()
                wcopy(l + 2, u).start()
        return carry
    lax.fori_loop(0, _LEN // 2, pair_body, 0)

    pltpu.sync_copy(acc_v, out_hbm.at[pl.ds(base, _SPW)])


_sc_embed = functools.partial(
    pl.kernel,
    out_type=jax.ShapeDtypeStruct((_BATCH, _NCLS, 16), jnp.float32),
    mesh=plsc.VectorSubcoreMesh(core_axis_name="c", subcore_axis_name="s"),
    scratch_types=[
        pltpu.VMEM((_LEN, _SPW), jnp.int32),
        pltpu.VMEM((2, _SPW, _DIM), jnp.float32),
        pltpu.VMEM((2, _NCLS, _DIM), jnp.float32),
        pltpu.VMEM((_SPW, _NCLS, 16), jnp.float32),
        pltpu.SemaphoreType.DMA,
        pltpu.SemaphoreType.DMA,
        pltpu.SemaphoreType.DMA,
        pltpu.SemaphoreType.DMA,
    ],
)(_sc_body)


def _logsoftmax_body(p_ref, b_ref, o_ref):
    p = p_ref[...]
    z0 = jnp.sum(p[:, :16], axis=1, keepdims=True)
    z1 = jnp.sum(p[:, 16:], axis=1, keepdims=True)
    z = jnp.concatenate([z0, z1], axis=1) + b_ref[...]
    m = jnp.max(z, axis=-1, keepdims=True)
    e = jnp.exp(z - m)
    lse = m + jnp.log(jnp.sum(e, axis=-1, keepdims=True))
    o_ref[...] = z - lse


def kernel(x, table, W, b):
    xT = x.T
    Wr = W.reshape(_NCLS, _LEN, _DIM)
    partials = _sc_embed(xT, table, Wr)
    return pl.pallas_call(
        _logsoftmax_body,
        out_shape=jax.ShapeDtypeStruct((_BATCH, _NCLS), jnp.float32),
    )(partials.reshape(_BATCH, _NCLS * 16), b.reshape(1, _NCLS))
